# initial kernel scaffold (unmeasured)
import jax
import jax.numpy as jnp
from jax import lax
from jax.experimental import pallas as pl
from jax.experimental.pallas import tpu as pltpu

N_DEV = 32
B = 2
SQ = 512
SKV = 512
H_PER = 8
DH = 64
D_MODEL = 768
ROWS = B * SQ
CH = ROWS // N_DEV


def _attn_body(x_ref, wq_ref, k_ref, v_ref, wo_ref, out_ref):
    h = pl.program_id(1)
    x2 = x_ref[0]
    q = jnp.dot(x2, wq_ref[...], preferred_element_type=jnp.float32)
    k2 = k_ref[0, :, 0, :]
    v2 = v_ref[0, :, 0, :]
    scores = jnp.dot(q, k2.T, preferred_element_type=jnp.float32) * 0.125
    qb = lax.broadcasted_iota(jnp.int32, (SQ, SKV), 0) // 64
    kb = lax.broadcasted_iota(jnp.int32, (SQ, SKV), 1) // 64
    mask = (qb % 4) == (kb % 4)
    scores = jnp.where(mask, scores, -1e9)
    m = jnp.max(scores, axis=-1, keepdims=True)
    w = jnp.exp(scores - m)
    w = w / jnp.sum(w, axis=-1, keepdims=True)
    ctx = jnp.dot(w, v2, preferred_element_type=jnp.float32)
    part = jnp.dot(ctx, wo_ref[...], preferred_element_type=jnp.float32)

    @pl.when(h == 0)
    def _():
        out_ref[0] = jnp.zeros_like(out_ref[0])

    out_ref[0] += part


def _ar_body(p_ref, out_ref, acc_ref, send1, recv1, send2, recv2):
    my = lax.axis_index("i")

    p1 = []
    for p in range(N_DEV):
        r = pltpu.make_async_remote_copy(
            src_ref=p_ref.at[pl.ds(p * CH, CH)],
            dst_ref=acc_ref.at[pl.ds(my * CH, CH)],
            send_sem=send1.at[p],
            recv_sem=recv1.at[my],
            device_id=(p,),
            device_id_type=pl.DeviceIdType.MESH,
        )
        p1.append(r)

        @pl.when(my != p)
        def _(r=r):
            r.start()

    for s in range(N_DEV):
        rcv = pltpu.make_async_remote_copy(
            src_ref=acc_ref.at[pl.ds(s * CH, CH)],
            dst_ref=acc_ref.at[pl.ds(s * CH, CH)],
            send_sem=send1.at[s],
            recv_sem=recv1.at[s],
            device_id=(s,),
            device_id_type=pl.DeviceIdType.MESH,
        )

        @pl.when(my != s)
        def _(rcv=rcv):
            rcv.wait_recv()

    own = p_ref[pl.ds(my * CH, CH), :]
    acc = acc_ref[...].reshape(N_DEV, CH, D_MODEL)
    sidx = lax.broadcasted_iota(jnp.int32, (N_DEV, 1, 1), 0)
    red = own + jnp.sum(jnp.where(sidx == my, 0.0, acc), axis=0)
    out_ref[pl.ds(my * CH, CH), :] = red

    p2 = []
    for p in range(N_DEV):
        r = pltpu.make_async_remote_copy(
            src_ref=out_ref.at[pl.ds(my * CH, CH)],
            dst_ref=out_ref.at[pl.ds(my * CH, CH)],
            send_sem=send2.at[p],
            recv_sem=recv2.at[my],
            device_id=(p,),
            device_id_type=pl.DeviceIdType.MESH,
        )
        p2.append(r)

        @pl.when(my != p)
        def _(r=r):
            r.start()

    for s in range(N_DEV):
        rcv = pltpu.make_async_remote_copy(
            src_ref=out_ref.at[pl.ds(s * CH, CH)],
            dst_ref=out_ref.at[pl.ds(s * CH, CH)],
            send_sem=send2.at[s],
            recv_sem=recv2.at[s],
            device_id=(s,),
            device_id_type=pl.DeviceIdType.MESH,
        )

        @pl.when(my != s)
        def _(rcv=rcv):
            rcv.wait_recv()

    for p in range(N_DEV):
        @pl.when(my != p)
        def _(r=p1[p]):
            r.wait_send()

        @pl.when(my != p)
        def _(r=p2[p]):
            r.wait_send()


def kernel(x, Wq, K_ext, V_ext, Wo):
    my = lax.axis_index("i")
    K_loc = lax.dynamic_slice_in_dim(K_ext, my * H_PER, H_PER, axis=2)
    V_loc = lax.dynamic_slice_in_dim(V_ext, my * H_PER, H_PER, axis=2)

    partial = pl.pallas_call(
        _attn_body,
        grid=(B, H_PER),
        in_specs=[
            pl.BlockSpec((1, SQ, D_MODEL), lambda b, h: (b, 0, 0)),
            pl.BlockSpec((D_MODEL, DH), lambda b, h: (0, h)),
            pl.BlockSpec((1, SKV, 1, DH), lambda b, h: (b, 0, h, 0)),
            pl.BlockSpec((1, SKV, 1, DH), lambda b, h: (b, 0, h, 0)),
            pl.BlockSpec((DH, D_MODEL), lambda b, h: (h, 0)),
        ],
        out_specs=pl.BlockSpec((1, SQ, D_MODEL), lambda b, h: (b, 0, 0)),
        out_shape=jax.ShapeDtypeStruct((B, SQ, D_MODEL), jnp.float32),
    )(x, Wq, K_loc, V_loc, Wo)

    out = pl.pallas_call(
        _ar_body,
        out_shape=jax.ShapeDtypeStruct((ROWS, D_MODEL), jnp.float32),
        in_specs=[pl.BlockSpec(memory_space=pltpu.VMEM)],
        out_specs=pl.BlockSpec(memory_space=pltpu.VMEM),
        scratch_shapes=[
            pltpu.VMEM((ROWS, D_MODEL), jnp.float32),
            pltpu.SemaphoreType.DMA((N_DEV,)),
            pltpu.SemaphoreType.DMA((N_DEV,)),
            pltpu.SemaphoreType.DMA((N_DEV,)),
            pltpu.SemaphoreType.DMA((N_DEV,)),
        ],
        compiler_params=pltpu.CompilerParams(collective_id=0),
    )(partial.reshape(ROWS, D_MODEL))
    return out.reshape(B, SQ, D_MODEL)


# baseline (device time: 213633 ns/iter reference)
import jax
import jax.numpy as jnp
from jax import lax
from jax.experimental import pallas as pl
from jax.experimental.pallas import tpu as pltpu

N_DEV = 32
B = 2
SQ = 512
SKV = 512
H_PER = 8
DH = 64
D_MODEL = 768
ROWS = B * SQ
CH = ROWS // N_DEV


def _attn_body(x_ref, wq_ref, k_ref, v_ref, wo_ref, out_ref):
    x2 = x_ref[0]
    q = jnp.dot(x2, wq_ref[...], preferred_element_type=jnp.float32)
    k = k_ref[0]
    v = v_ref[0]
    qb = lax.broadcasted_iota(jnp.int32, (SQ, SKV), 0) // 64
    kb = lax.broadcasted_iota(jnp.int32, (SQ, SKV), 1) // 64
    mask = (qb % 4) == (kb % 4)
    ctxs = []
    for h in range(H_PER):
        sl = slice(h * DH, (h + 1) * DH)
        scores = jnp.dot(q[:, sl], k[:, sl].T,
                         preferred_element_type=jnp.float32) * 0.125
        scores = jnp.where(mask, scores, -1e9)
        m = jnp.max(scores, axis=-1, keepdims=True)
        w = jnp.exp(scores - m)
        w = w / jnp.sum(w, axis=-1, keepdims=True)
        ctxs.append(jnp.dot(w, v[:, sl], preferred_element_type=jnp.float32))
    ctx = jnp.concatenate(ctxs, axis=1)
    out_ref[0] = jnp.dot(ctx, wo_ref[...], preferred_element_type=jnp.float32)


def _ar_body(p_ref, out_ref, acc_ref, send1, recv1, send2, recv2):
    my = lax.axis_index("i")

    p1 = []
    for p in range(N_DEV):
        r = pltpu.make_async_remote_copy(
            src_ref=p_ref.at[pl.ds(p * CH, CH)],
            dst_ref=acc_ref.at[pl.ds(my * CH, CH)],
            send_sem=send1.at[p],
            recv_sem=recv1.at[my],
            device_id=(p,),
            device_id_type=pl.DeviceIdType.MESH,
        )
        p1.append(r)

        @pl.when(my != p)
        def _(r=r):
            r.start()

    for s in range(N_DEV):
        rcv = pltpu.make_async_remote_copy(
            src_ref=acc_ref.at[pl.ds(s * CH, CH)],
            dst_ref=acc_ref.at[pl.ds(s * CH, CH)],
            send_sem=send1.at[s],
            recv_sem=recv1.at[s],
            device_id=(s,),
            device_id_type=pl.DeviceIdType.MESH,
        )

        @pl.when(my != s)
        def _(rcv=rcv):
            rcv.wait_recv()

    own = p_ref[pl.ds(my * CH, CH), :]
    acc = acc_ref[...].reshape(N_DEV, CH, D_MODEL)
    sidx = lax.broadcasted_iota(jnp.int32, (N_DEV, 1, 1), 0)
    red = own + jnp.sum(jnp.where(sidx == my, 0.0, acc), axis=0)
    out_ref[pl.ds(my * CH, CH), :] = red

    p2 = []
    for p in range(N_DEV):
        r = pltpu.make_async_remote_copy(
            src_ref=out_ref.at[pl.ds(my * CH, CH)],
            dst_ref=out_ref.at[pl.ds(my * CH, CH)],
            send_sem=send2.at[p],
            recv_sem=recv2.at[my],
            device_id=(p,),
            device_id_type=pl.DeviceIdType.MESH,
        )
        p2.append(r)

        @pl.when(my != p)
        def _(r=r):
            r.start()

    for s in range(N_DEV):
        rcv = pltpu.make_async_remote_copy(
            src_ref=out_ref.at[pl.ds(s * CH, CH)],
            dst_ref=out_ref.at[pl.ds(s * CH, CH)],
            send_sem=send2.at[s],
            recv_sem=recv2.at[s],
            device_id=(s,),
            device_id_type=pl.DeviceIdType.MESH,
        )

        @pl.when(my != s)
        def _(rcv=rcv):
            rcv.wait_recv()

    for p in range(N_DEV):
        @pl.when(my != p)
        def _(r=p1[p]):
            r.wait_send()

        @pl.when(my != p)
        def _(r=p2[p]):
            r.wait_send()


def kernel(x, Wq, K_ext, V_ext, Wo):
    my = lax.axis_index("i")
    K_loc = lax.dynamic_slice_in_dim(K_ext, my * H_PER, H_PER, axis=2)
    V_loc = lax.dynamic_slice_in_dim(V_ext, my * H_PER, H_PER, axis=2)

    HD = H_PER * DH
    partial = pl.pallas_call(
        _attn_body,
        grid=(B,),
        in_specs=[
            pl.BlockSpec((1, SQ, D_MODEL), lambda b: (b, 0, 0)),
            pl.BlockSpec((D_MODEL, HD), lambda b: (0, 0)),
            pl.BlockSpec((1, SKV, HD), lambda b: (b, 0, 0)),
            pl.BlockSpec((1, SKV, HD), lambda b: (b, 0, 0)),
            pl.BlockSpec((HD, D_MODEL), lambda b: (0, 0)),
        ],
        out_specs=pl.BlockSpec((1, SQ, D_MODEL), lambda b: (b, 0, 0)),
        out_shape=jax.ShapeDtypeStruct((B, SQ, D_MODEL), jnp.float32),
    )(x, Wq, K_loc.reshape(B, SKV, HD), V_loc.reshape(B, SKV, HD), Wo)

    out = pl.pallas_call(
        _ar_body,
        out_shape=jax.ShapeDtypeStruct((ROWS, D_MODEL), jnp.float32),
        in_specs=[pl.BlockSpec(memory_space=pltpu.VMEM)],
        out_specs=pl.BlockSpec(memory_space=pltpu.VMEM),
        scratch_shapes=[
            pltpu.VMEM((ROWS, D_MODEL), jnp.float32),
            pltpu.SemaphoreType.DMA((N_DEV,)),
            pltpu.SemaphoreType.DMA((N_DEV,)),
            pltpu.SemaphoreType.DMA((N_DEV,)),
            pltpu.SemaphoreType.DMA((N_DEV,)),
        ],
    )(partial.reshape(ROWS, D_MODEL))
    return out.reshape(B, SQ, D_MODEL)
